# baseline (speedup 1.0000x reference)
import functools

import jax
import jax.numpy as jnp
from jax import lax
from jax.experimental import pallas as pl
from jax.experimental.pallas import tpu as pltpu
from jax.experimental.pallas import tpu_sc as plsc

C = 19
SUP = 2048
SMALL = 8192
SUP2 = SUP + 16
SMALL2 = SMALL + 16
L = 16
NTILES = 32
CH = 2048
TCB = 8192
GRP = 16 * L


def _prep_body(dead, x_ref, m_ref, i_ref, o_ref, oi_ref):
    x = x_ref[0]
    mask = m_ref[0, 0]
    ids = i_ref[0, 0]
    mx = jnp.max(x, axis=0, keepdims=True)
    lse = jnp.log(jnp.sum(jnp.exp(x - mx), axis=0, keepdims=True))
    mlse = jnp.where(mask > 0, mx + lse, 0.0)
    o_ref[0] = mlse - jnp.where(mask > 0, x, 1.0)
    lane = lax.broadcasted_iota(jnp.int32, ids.shape, 1) & (L - 1)
    oi_ref[0, 0] = jnp.where(mask > 0, ids, dead + lane)


def _prep(x, mask, ids, dead, blk):
    n, c, p = x.shape
    nb = p // blk
    mask4 = mask.reshape(n, nb, 1, blk)
    ids4 = ids.reshape(n, nb, 1, blk)
    out, out_ids = pl.pallas_call(
        functools.partial(_prep_body, dead),
        grid=(n, nb),
        in_specs=[
            pl.BlockSpec((1, c, blk), lambda i, j: (i, 0, j)),
            pl.BlockSpec((1, 1, 1, blk), lambda i, j: (i, j, 0, 0)),
            pl.BlockSpec((1, 1, 1, blk), lambda i, j: (i, j, 0, 0)),
        ],
        out_specs=[
            pl.BlockSpec((1, c, blk), lambda i, j: (i, 0, j)),
            pl.BlockSpec((1, 1, 1, blk), lambda i, j: (i, j, 0, 0)),
        ],
        out_shape=[
            jax.ShapeDtypeStruct((n, c, p), jnp.float32),
            jax.ShapeDtypeStruct((n, nb, 1, blk), jnp.int32),
        ],
    )(x, mask4, ids4)
    return out, out_ids.reshape(n, p)


def _flags_body(i_ref, o_ref):
    a = i_ref[0]
    dup = jnp.zeros(a.shape[1:], jnp.bool_)[None]
    for k in range(1, 9):
        dup = jnp.logical_or(dup, jnp.any(a == jnp.roll(a, k, axis=0),
                                          axis=0, keepdims=True))
    o_ref[0, 0] = dup.astype(jnp.int32)


def _flags(ids2, blk):
    n, p = ids2.shape
    g = p // L
    nb = g // blk
    ids_t = ids2.reshape(n, g, L).transpose(0, 2, 1)
    out = pl.pallas_call(
        _flags_body,
        grid=(n, nb),
        in_specs=[pl.BlockSpec((1, L, blk), lambda i, j: (i, 0, j))],
        out_specs=pl.BlockSpec((1, 1, 1, blk), lambda i, j: (i, j, 0, 0)),
        out_shape=jax.ShapeDtypeStruct((n, nb, 1, blk), jnp.int32),
    )(ids_t)
    return out.reshape(n, g)


def _fill_f32(ref, n, val):
    v = jnp.full((L,), val, jnp.float32)

    def body(i, _):
        ref[pl.ds(i * L, L)] = v
        return 0

    lax.fori_loop(0, n // L, body, 0)


def _fill_i32(ref, n, val):
    v = jnp.full((L,), val, jnp.int32)

    def body(i, _):
        ref[pl.ds(i * L, L)] = v
        return 0

    lax.fori_loop(0, n // L, body, 0)


def _scatter_min_retry(ref, idx, v, m0):
    def cond(st):
        return jnp.max(st[0].astype(jnp.int32)) > 0

    def body(st):
        pend, vv = st
        old = plsc.load_gather(ref, [idx])
        new = jnp.minimum(old, vv)
        plsc.store_scatter(ref, [idx], new, mask=pend)
        chk = plsc.load_gather(ref, [idx])
        return jnp.logical_and(pend, chk > new), vv

    lax.while_loop(cond, body, (m0, v))


def _make_sc1(n, ps, pw):
    n_strong = n * C
    n_weak = n * C
    n_tasks = n_strong + n_weak + n
    k_max = (n_tasks + NTILES - 1) // NTILES
    mesh = plsc.VectorSubcoreMesh(core_axis_name="c", subcore_axis_name="s")

    @functools.partial(
        pl.kernel,
        mesh=mesh,
        compiler_params=pltpu.CompilerParams(needs_layout_passes=False),
        out_type=(
            jax.ShapeDtypeStruct((n, C, SMALL2), jnp.float32),
            jax.ShapeDtypeStruct((n, C, SMALL2), jnp.float32),
            jax.ShapeDtypeStruct((n, SMALL2), jnp.float32),
            jax.ShapeDtypeStruct((n, C, SUP2), jnp.int32),
        ),
    scratch_types=(
            pltpu.VMEM((2 * CH,), jnp.float32),
            pltpu.VMEM((2 * CH,), jnp.int32),
            pltpu.VMEM((2 * (CH // L),), jnp.int32),
            pltpu.VMEM((pw,), jnp.float32),
            pltpu.VMEM((SMALL2,), jnp.float32),
            pltpu.VMEM((SMALL2,), jnp.float32),
            pltpu.VMEM((SMALL2,), jnp.float32),
            pltpu.VMEM((SMALL2,), jnp.float32),
            pltpu.VMEM((SMALL2,), jnp.float32),
            pltpu.VMEM((SUP2,), jnp.float32),
            pltpu.VMEM((SUP2,), jnp.float32),
            pltpu.VMEM((SUP2,), jnp.int32),
            pltpu.VMEM((SUP2,), jnp.int32),
            pltpu.SemaphoreType.DMA,
            pltpu.SemaphoreType.DMA,
            pltpu.SemaphoreType.DMA,
        ),
    )
    def sc1(xs, idss, fls, xw, supw, flw,
            sum_hbm, max_hbm, sizes_hbm, amin_hbm,
            xb, ib, fb, pcache, sumt, maxta, maxtb, maxtc, maxtd,
            wmaxta, wmaxtb, waminta, wamintb,
            sx, si, sf):
        wid = lax.axis_index("s") * 2 + lax.axis_index("c")

        def xcopy(src, c, half, buf, sem, width):
            return pltpu.make_async_copy(
                src.at[pl.ds(c * width, width)],
                buf.at[pl.ds(pl.multiple_of(half * width, 8), width)],
                sem)

        def stream_loop(nch, issue, wait, proc, scat_issue=None,
                        scat_wait=None):
            issue(0, 0)

            def cbody(c, _):
                par = lax.rem(c, 2)

                if scat_wait is not None:
                    @pl.when(c > 0)
                    def _():
                        scat_wait()

                @pl.when(c + 1 < nch)
                def _():
                    issue(c + 1, 1 - par)

                wait(c, par)
                if scat_issue is not None:
                    scat_issue(par)
                proc(c, par)
                return 0

            lax.fori_loop(0, nch, cbody, 0)
            if scat_wait is not None:
                scat_wait()

        def merge_min_f(ta, tb, nrows):
            def body(i, _):
                sl = pl.ds(i * L, L)
                ta[sl] = jnp.minimum(ta[sl], tb[sl])
                return 0

            lax.fori_loop(0, nrows // L, body, 0)

        def strong_task(t):
            img = t // C
            cls = t - img * C
            _fill_f32(sumt, SMALL2, 0.0)
            _fill_f32(maxta, SMALL2, 1e30)
            _fill_f32(maxtb, SMALL2, 1e30)
            _fill_f32(maxtc, SMALL2, 1e30)
            _fill_f32(maxtd, SMALL2, 1e30)
            xrow = xs.at[img, cls]
            irow = idss.at[img]
            frow = fls.at[img]

            def issue(c, half):
                xcopy(xrow, c, half, xb, sx, CH).start()
                xcopy(irow, c, half, ib, si, CH).start()
                xcopy(frow, c, half, fb, sf, CH // L).start()

            def wait(c, half):
                xcopy(xrow, c, half, xb, sx, CH).wait()
                xcopy(irow, c, half, ib, si, CH).wait()
                xcopy(frow, c, half, fb, sf, CH // L).wait()

            def vstep(base, v):
                tbl = (maxta, maxtb, maxtc, maxtd)[v % 4]
                sl = pl.ds(base + v * L, L)
                x = xb[sl]
                idx = ib[sl]
                plsc.addupdate_scatter(sumt, [idx], x)
                old = plsc.load_gather(tbl, [idx])
                plsc.store_scatter(tbl, [idx], jnp.minimum(old, x))
                return tbl, idx, x

            def proc(c, par):
                def group(g, _):
                    fl = fb[pl.ds(par * (CH // L) + g * L, L)]
                    base = par * CH + g * GRP

                    @pl.when(jnp.max(fl) == 0)
                    def _():
                        for v in range(L):
                            vstep(base, v)

                    @pl.when(jnp.max(fl) != 0)
                    def _():
                        for v in range(L):
                            tbl, idx, x = vstep(base, v)
                            _scatter_min_retry(tbl, idx, x,
                                               jnp.ones((L,), jnp.bool_))

                    return 0

                lax.fori_loop(0, CH // GRP, group, 0)

            stream_loop(ps // CH, issue, wait, proc)
            merge_min_f(maxtb, maxtd, SMALL2)
            merge_min_f(maxta, maxtc, SMALL2)
            merge_min_f(maxta, maxtb, SMALL2)
            pltpu.sync_copy(sumt, sum_hbm.at[img, cls])
            pltpu.sync_copy(maxta, max_hbm.at[img, cls])

        def weak_task(t):
            img = t // C
            cls = t - img * C
            _fill_f32(wmaxta, SUP2, 1e30)
            _fill_f32(wmaxtb, SUP2, 1e30)
            _fill_i32(waminta, SUP2, pw)
            _fill_i32(wamintb, SUP2, pw)
            xrow = xw.at[img, cls]
            irow = supw.at[img]
            frow = flw.at[img]

            def issue_a(c, half):
                xcopy(xrow, c, half, xb, sx, CH).start()
                xcopy(irow, c, half, ib, si, CH).start()
                xcopy(frow, c, half, fb, sf, CH // L).start()

            def wait_a(c, half):
                xcopy(xrow, c, half, xb, sx, CH).wait()
                xcopy(irow, c, half, ib, si, CH).wait()
                xcopy(frow, c, half, fb, sf, CH // L).wait()

            def vstep_a(coff, base, v):
                tbl = wmaxta if v % 2 == 0 else wmaxtb
                sl = pl.ds(base + v * L, L)
                x = xb[sl]
                idx = ib[sl]
                pcache[pl.ds(coff + v * L, L)] = x
                old = plsc.load_gather(tbl, [idx])
                plsc.store_scatter(tbl, [idx], jnp.minimum(old, x))
                return tbl, idx, x

            def proc_a(c, par):
                def group(g, _):
                    fl = fb[pl.ds(par * (CH // L) + g * L, L)]
                    base = par * CH + g * GRP
                    coff = c * CH + g * GRP

                    @pl.when(jnp.max(fl) == 0)
                    def _():
                        for v in range(L):
                            vstep_a(coff, base, v)

                    @pl.when(jnp.max(fl) != 0)
                    def _():
                        for v in range(L):
                            tbl, idx, x = vstep_a(coff, base, v)
                            _scatter_min_retry(tbl, idx, x,
                                               jnp.ones((L,), jnp.bool_))

                    return 0

                lax.fori_loop(0, CH // GRP, group, 0)

            stream_loop(pw // CH, issue_a, wait_a, proc_a)
            merge_min_f(wmaxta, wmaxtb, SUP2)

            def issue_b(c, half):
                xcopy(irow, c, half, ib, si, CH).start()
                xcopy(frow, c, half, fb, sf, CH // L).start()

            def wait_b(c, half):
                xcopy(irow, c, half, ib, si, CH).wait()
                xcopy(frow, c, half, fb, sf, CH // L).wait()

            def vstep_b(coff, base, v):
                tbl = waminta if v % 2 == 0 else wamintb
                x = pcache[pl.ds(coff + v * L, L)]
                idx = ib[pl.ds(base + v * L, L)]
                smin = plsc.load_gather(wmaxta, [idx])
                hit = x == smin
                gidx = coff + v * L + lax.iota(jnp.int32, L)
                return tbl, idx, gidx, hit

            def proc_b(c, par):
                def group(g, _):
                    fl = fb[pl.ds(par * (CH // L) + g * L, L)]
                    base = par * CH + g * GRP
                    coff = c * CH + g * GRP

                    @pl.when(jnp.max(fl) == 0)
                    def _():
                        for v in range(L):
                            tbl, idx, gidx, hit = vstep_b(coff, base, v)
                            old = plsc.load_gather(tbl, [idx])
                            plsc.store_scatter(
                                tbl, [idx], jnp.minimum(old, gidx),
                                mask=hit)

                    @pl.when(jnp.max(fl) != 0)
                    def _():
                        for v in range(L):
                            tbl, idx, gidx, hit = vstep_b(coff, base, v)
                            _scatter_min_retry(tbl, idx, gidx, hit)

                    return 0

                lax.fori_loop(0, CH // GRP, group, 0)

            stream_loop(pw // CH, issue_b, wait_b, proc_b)

            def merge_min(i, _):
                sl = pl.ds(i * L, L)
                waminta[sl] = jnp.minimum(waminta[sl], wamintb[sl])
                return 0

            lax.fori_loop(0, SUP2 // L, merge_min, 0)
            pltpu.sync_copy(waminta, amin_hbm.at[img, cls])

        def sizes_task(img):
            _fill_f32(sumt, SMALL2, 0.0)
            irow = idss.at[img]
            one = jnp.full((L,), 1.0, jnp.float32)

            def issue(c, half):
                xcopy(irow, c, half, ib, si, CH).start()

            def wait(c, half):
                xcopy(irow, c, half, ib, si, CH).wait()

            def proc(c, par):
                def grp(g, _):
                    base = par * CH + g * GRP
                    for v in range(L):
                        idx = ib[pl.ds(base + v * L, L)]
                        plsc.addupdate_scatter(sumt, [idx], one)
                    return 0

                lax.fori_loop(0, CH // GRP, grp, 0)

            stream_loop(ps // CH, issue, wait, proc)
            pltpu.sync_copy(sumt, sizes_hbm.at[img])

        def kloop(k, _):
            task = wid + k * NTILES

            @pl.when(task < n_strong)
            def _():
                strong_task(task)

            @pl.when(jnp.logical_and(task >= n_strong,
                                     task < n_strong + n_weak))
            def _():
                weak_task(task - n_strong)

            @pl.when(jnp.logical_and(task >= n_strong + n_weak,
                                     task < n_tasks))
            def _():
                sizes_task(task - n_strong - n_weak)

            return 0

        lax.fori_loop(0, k_max, kloop, 0)

    return sc1


def _make_sc2(n, pw):
    n_tasks = n * C
    k_max = (n_tasks + NTILES - 1) // NTILES
    mesh = plsc.VectorSubcoreMesh(core_axis_name="c", subcore_axis_name="s")

    @functools.partial(
        pl.kernel,
        mesh=mesh,
        compiler_params=pltpu.CompilerParams(needs_layout_passes=False),
        out_type=(
            jax.ShapeDtypeStruct((n, C, L), jnp.float32),
            jax.ShapeDtypeStruct((n, C, L), jnp.float32),
        ),
        scratch_types=(
            pltpu.VMEM((pw,), jnp.int32),
            pltpu.VMEM((SMALL2,), jnp.float32),
            pltpu.VMEM((SMALL2,), jnp.float32),
            pltpu.VMEM((SMALL2,), jnp.float32),
            pltpu.VMEM((SUP2,), jnp.int32),
            pltpu.VMEM((SUP2,), jnp.int32),
            pltpu.VMEM((SUP,), jnp.int32),
            pltpu.VMEM((L,), jnp.float32),
            pltpu.VMEM((L,), jnp.float32),
        ),
    )
    def sc2(sum_hbm, max_hbm, sizes_hbm, amin_hbm, smallw_hbm, trg_hbm,
            lout, nout,
            swv, sumt, maxt, sizest, am0, amc, tgc, lbuf, nbuf):
        wid = lax.axis_index("s") * 2 + lax.axis_index("c")

        def task_body(t):
            img = t // C
            cls = t - img * C
            pltpu.sync_copy(smallw_hbm.at[img], swv)
            pltpu.sync_copy(sum_hbm.at[img, cls], sumt)
            pltpu.sync_copy(max_hbm.at[img, cls], maxt)
            pltpu.sync_copy(sizes_hbm.at[img], sizest)
            pltpu.sync_copy(amin_hbm.at[img, 0], am0)
            pltpu.sync_copy(amin_hbm.at[img, cls], amc)
            pltpu.sync_copy(trg_hbm.at[img, cls], tgc)

            def vbody(v, st):
                lacc, nacc = st
                sl = pl.ds(v * L, L)
                a0 = am0[sl]
                ac = amc[sl]
                tg = tgc[sl]
                pm = jnp.logical_and(a0 < pw, tg > 0)
                sel = plsc.load_gather(swv, [jnp.minimum(ac, pw - 1)])
                val = plsc.load_gather(sumt, [sel])
                w = jnp.exp(-plsc.load_gather(maxt, [sel]))
                sz = plsc.load_gather(sizest, [sel])
                lacc = lacc + jnp.where(pm, w * val, 0.0)
                nacc = nacc + jnp.where(pm, sz, 0.0)
                return lacc, nacc

            z = jnp.zeros((L,), jnp.float32)
            lacc, nacc = lax.fori_loop(0, SUP // L, vbody, (z, z))
            lbuf[...] = lacc
            nbuf[...] = nacc
            pltpu.sync_copy(lbuf, lout.at[img, cls])
            pltpu.sync_copy(nbuf, nout.at[img, cls])

        def kloop(k, _):
            task = wid + k * NTILES

            @pl.when(task < n_tasks)
            def _():
                task_body(task)

            return 0

        lax.fori_loop(0, k_max, kloop, 0)

    return sc2


def kernel(inputs, inputs_weak, targets, spmasks, spmasks_weak,
           superpixels, superpixels_weak, superpixel_smalls, spx_smalls_weak):
    n, c, h, w = inputs.shape
    _, _, hw, ww = inputs_weak.shape
    ps = h * w
    pw = hw * ww

    xs = inputs.reshape(n, c, ps)
    xw = inputs_weak.reshape(n, c, pw)
    mask_s = spmasks.reshape(n, ps).astype(jnp.int32)
    mask_w = spmasks_weak.reshape(n, pw).astype(jnp.int32)
    ids_s = superpixel_smalls.reshape(n, ps).astype(jnp.int32)
    sup_w = superpixels_weak.reshape(n, pw).astype(jnp.int32)
    small_w = spx_smalls_weak.reshape(n, pw).astype(jnp.int32)
    trg = targets[:, :, :C].transpose(0, 2, 1).astype(jnp.int32)

    enc_s, ids2_s = _prep(xs, mask_s, ids_s, SMALL, TCB)
    enc_w, ids2_w = _prep(xw, mask_w, sup_w, SUP, TCB)
    flags_s = _flags(ids2_s, CH)
    flags_w = _flags(ids2_w, CH)

    sum_t, max_t, sizes_t, amin_t = _make_sc1(n, ps, pw)(
        enc_s, ids2_s, flags_s, enc_w, ids2_w, flags_w)
    lparts, nparts = _make_sc2(n, pw)(
        sum_t, max_t, sizes_t, amin_t, small_w, trg)

    loss = jnp.sum(lparts)
    num_valid = jnp.float32(1.0) + jnp.sum(nparts)
    return loss / num_valid


# speedup vs baseline: 1.0253x; 1.0253x over previous
import functools

import jax
import jax.numpy as jnp
from jax import lax
from jax.experimental import pallas as pl
from jax.experimental.pallas import tpu as pltpu
from jax.experimental.pallas import tpu_sc as plsc

C = 19
SUP = 2048
SMALL = 8192
SUP2 = SUP + 16
SMALL2 = SMALL + 16
L = 16
NTILES = 32
CH = 4096
TCB = 8192
GRP = 16 * L


def _prep_body(dead, x_ref, m_ref, i_ref, o_ref, oi_ref):
    x = x_ref[0]
    mask = m_ref[0, 0]
    ids = i_ref[0, 0]
    mx = jnp.max(x, axis=0, keepdims=True)
    lse = jnp.log(jnp.sum(jnp.exp(x - mx), axis=0, keepdims=True))
    mlse = jnp.where(mask > 0, mx + lse, 0.0)
    o_ref[0] = mlse - jnp.where(mask > 0, x, 1.0)
    lane = lax.broadcasted_iota(jnp.int32, ids.shape, 1) & (L - 1)
    oi_ref[0, 0] = jnp.where(mask > 0, ids, dead + lane)


def _prep(x, mask, ids, dead, blk):
    n, c, p = x.shape
    nb = p // blk
    mask4 = mask.reshape(n, nb, 1, blk)
    ids4 = ids.reshape(n, nb, 1, blk)
    out, out_ids = pl.pallas_call(
        functools.partial(_prep_body, dead),
        grid=(n, nb),
        in_specs=[
            pl.BlockSpec((1, c, blk), lambda i, j: (i, 0, j)),
            pl.BlockSpec((1, 1, 1, blk), lambda i, j: (i, j, 0, 0)),
            pl.BlockSpec((1, 1, 1, blk), lambda i, j: (i, j, 0, 0)),
        ],
        out_specs=[
            pl.BlockSpec((1, c, blk), lambda i, j: (i, 0, j)),
            pl.BlockSpec((1, 1, 1, blk), lambda i, j: (i, j, 0, 0)),
        ],
        out_shape=[
            jax.ShapeDtypeStruct((n, c, p), jnp.float32),
            jax.ShapeDtypeStruct((n, nb, 1, blk), jnp.int32),
        ],
    )(x, mask4, ids4)
    return out, out_ids.reshape(n, p)


def _flags_body(i_ref, o_ref):
    a = i_ref[0]
    dup = jnp.zeros(a.shape[1:], jnp.bool_)[None]
    for k in range(1, 9):
        dup = jnp.logical_or(dup, jnp.any(a == jnp.roll(a, k, axis=0),
                                          axis=0, keepdims=True))
    o_ref[0, 0] = dup.astype(jnp.int32)


def _flags(ids2, blk):
    n, p = ids2.shape
    g = p // L
    nb = g // blk
    ids_t = ids2.reshape(n, g, L).transpose(0, 2, 1)
    out = pl.pallas_call(
        _flags_body,
        grid=(n, nb),
        in_specs=[pl.BlockSpec((1, L, blk), lambda i, j: (i, 0, j))],
        out_specs=pl.BlockSpec((1, 1, 1, blk), lambda i, j: (i, j, 0, 0)),
        out_shape=jax.ShapeDtypeStruct((n, nb, 1, blk), jnp.int32),
    )(ids_t)
    return out.reshape(n, g)


def _fill_f32(ref, n, val):
    v = jnp.full((L,), val, jnp.float32)

    def body(i, _):
        ref[pl.ds(i * L, L)] = v
        return 0

    lax.fori_loop(0, n // L, body, 0)


def _fill_i32(ref, n, val):
    v = jnp.full((L,), val, jnp.int32)

    def body(i, _):
        ref[pl.ds(i * L, L)] = v
        return 0

    lax.fori_loop(0, n // L, body, 0)


def _scatter_min_retry(ref, idx, v, m0):
    def cond(st):
        return jnp.max(st[0].astype(jnp.int32)) > 0

    def body(st):
        pend, vv = st
        old = plsc.load_gather(ref, [idx])
        new = jnp.minimum(old, vv)
        plsc.store_scatter(ref, [idx], new, mask=pend)
        chk = plsc.load_gather(ref, [idx])
        return jnp.logical_and(pend, chk > new), vv

    lax.while_loop(cond, body, (m0, v))


def _make_sc1(n, ps, pw):
    n_strong = n * C
    n_weak = n * C
    n_tasks = n_strong + n_weak + n
    k_max = (n_tasks + NTILES - 1) // NTILES
    mesh = plsc.VectorSubcoreMesh(core_axis_name="c", subcore_axis_name="s")

    @functools.partial(
        pl.kernel,
        mesh=mesh,
        compiler_params=pltpu.CompilerParams(needs_layout_passes=False),
        out_type=(
            jax.ShapeDtypeStruct((n, C, SMALL2), jnp.float32),
            jax.ShapeDtypeStruct((n, C, SMALL2), jnp.float32),
            jax.ShapeDtypeStruct((n, SMALL2), jnp.float32),
            jax.ShapeDtypeStruct((n, C, SUP2), jnp.int32),
        ),
    scratch_types=(
            pltpu.VMEM((2 * CH,), jnp.float32),
            pltpu.VMEM((2 * CH,), jnp.int32),
            pltpu.VMEM((2 * (CH // L),), jnp.int32),
            pltpu.VMEM((pw,), jnp.float32),
            pltpu.VMEM((SMALL2,), jnp.float32),
            pltpu.VMEM((SMALL2,), jnp.float32),
            pltpu.VMEM((SMALL2,), jnp.float32),
            pltpu.VMEM((SUP2,), jnp.float32),
            pltpu.VMEM((SUP2,), jnp.float32),
            pltpu.VMEM((SUP2,), jnp.int32),
            pltpu.VMEM((SUP2,), jnp.int32),
            pltpu.SemaphoreType.DMA,
            pltpu.SemaphoreType.DMA,
            pltpu.SemaphoreType.DMA,
        ),
    )
    def sc1(xs, idss, fls, xw, supw, flw,
            sum_hbm, max_hbm, sizes_hbm, amin_hbm,
            xb, ib, fb, pcache, sumt, maxta, maxtb,
            wmaxta, wmaxtb, waminta, wamintb,
            sx, si, sf):
        wid = lax.axis_index("s") * 2 + lax.axis_index("c")

        def xcopy(src, c, half, buf, sem, width):
            return pltpu.make_async_copy(
                src.at[pl.ds(c * width, width)],
                buf.at[pl.ds(pl.multiple_of(half * width, 8), width)],
                sem)

        def stream_loop(nch, issue, wait, proc, scat_issue=None,
                        scat_wait=None):
            issue(0, 0)

            def cbody(c, _):
                par = lax.rem(c, 2)

                if scat_wait is not None:
                    @pl.when(c > 0)
                    def _():
                        scat_wait()

                @pl.when(c + 1 < nch)
                def _():
                    issue(c + 1, 1 - par)

                wait(c, par)
                if scat_issue is not None:
                    scat_issue(par)
                proc(c, par)
                return 0

            lax.fori_loop(0, nch, cbody, 0)
            if scat_wait is not None:
                scat_wait()

        def merge_min_f(ta, tb, nrows):
            def body(i, _):
                sl = pl.ds(i * L, L)
                ta[sl] = jnp.minimum(ta[sl], tb[sl])
                return 0

            lax.fori_loop(0, nrows // L, body, 0)

        def strong_task(t):
            img = t // C
            cls = t - img * C
            _fill_f32(sumt, SMALL2, 0.0)
            _fill_f32(maxta, SMALL2, 1e30)
            _fill_f32(maxtb, SMALL2, 1e30)
            xrow = xs.at[img, cls]
            irow = idss.at[img]
            frow = fls.at[img]

            def issue(c, half):
                xcopy(xrow, c, half, xb, sx, CH).start()
                xcopy(irow, c, half, ib, si, CH).start()
                xcopy(frow, c, half, fb, sf, CH // L).start()

            def wait(c, half):
                xcopy(xrow, c, half, xb, sx, CH).wait()
                xcopy(irow, c, half, ib, si, CH).wait()
                xcopy(frow, c, half, fb, sf, CH // L).wait()

            def vstep(base, v):
                tbl = maxta if v % 2 == 0 else maxtb
                sl = pl.ds(base + v * L, L)
                x = xb[sl]
                idx = ib[sl]
                plsc.addupdate_scatter(sumt, [idx], x)
                old = plsc.load_gather(tbl, [idx])
                plsc.store_scatter(tbl, [idx], jnp.minimum(old, x))
                return tbl, idx, x

            def proc(c, par):
                def group(g, _):
                    fl = fb[pl.ds(par * (CH // L) + g * L, L)]
                    base = par * CH + g * GRP

                    @pl.when(jnp.max(fl) == 0)
                    def _():
                        for v in range(L):
                            vstep(base, v)

                    @pl.when(jnp.max(fl) != 0)
                    def _():
                        for v in range(L):
                            tbl, idx, x = vstep(base, v)
                            _scatter_min_retry(tbl, idx, x,
                                               jnp.ones((L,), jnp.bool_))

                    return 0

                lax.fori_loop(0, CH // GRP, group, 0)

            stream_loop(ps // CH, issue, wait, proc)
            merge_min_f(maxta, maxtb, SMALL2)
            pltpu.sync_copy(sumt, sum_hbm.at[img, cls])
            pltpu.sync_copy(maxta, max_hbm.at[img, cls])

        def weak_task(t):
            img = t // C
            cls = t - img * C
            _fill_f32(wmaxta, SUP2, 1e30)
            _fill_f32(wmaxtb, SUP2, 1e30)
            _fill_i32(waminta, SUP2, pw)
            _fill_i32(wamintb, SUP2, pw)
            xrow = xw.at[img, cls]
            irow = supw.at[img]
            frow = flw.at[img]

            def issue_a(c, half):
                xcopy(xrow, c, half, xb, sx, CH).start()
                xcopy(irow, c, half, ib, si, CH).start()
                xcopy(frow, c, half, fb, sf, CH // L).start()

            def wait_a(c, half):
                xcopy(xrow, c, half, xb, sx, CH).wait()
                xcopy(irow, c, half, ib, si, CH).wait()
                xcopy(frow, c, half, fb, sf, CH // L).wait()

            def vstep_a(coff, base, v):
                tbl = wmaxta if v % 2 == 0 else wmaxtb
                sl = pl.ds(base + v * L, L)
                x = xb[sl]
                idx = ib[sl]
                pcache[pl.ds(coff + v * L, L)] = x
                old = plsc.load_gather(tbl, [idx])
                plsc.store_scatter(tbl, [idx], jnp.minimum(old, x))
                return tbl, idx, x

            def proc_a(c, par):
                def group(g, _):
                    fl = fb[pl.ds(par * (CH // L) + g * L, L)]
                    base = par * CH + g * GRP
                    coff = c * CH + g * GRP

                    @pl.when(jnp.max(fl) == 0)
                    def _():
                        for v in range(L):
                            vstep_a(coff, base, v)

                    @pl.when(jnp.max(fl) != 0)
                    def _():
                        for v in range(L):
                            tbl, idx, x = vstep_a(coff, base, v)
                            _scatter_min_retry(tbl, idx, x,
                                               jnp.ones((L,), jnp.bool_))

                    return 0

                lax.fori_loop(0, CH // GRP, group, 0)

            stream_loop(pw // CH, issue_a, wait_a, proc_a)
            merge_min_f(wmaxta, wmaxtb, SUP2)

            def issue_b(c, half):
                xcopy(irow, c, half, ib, si, CH).start()
                xcopy(frow, c, half, fb, sf, CH // L).start()

            def wait_b(c, half):
                xcopy(irow, c, half, ib, si, CH).wait()
                xcopy(frow, c, half, fb, sf, CH // L).wait()

            def vstep_b(coff, base, v):
                tbl = waminta if v % 2 == 0 else wamintb
                x = pcache[pl.ds(coff + v * L, L)]
                idx = ib[pl.ds(base + v * L, L)]
                smin = plsc.load_gather(wmaxta, [idx])
                hit = x == smin
                gidx = coff + v * L + lax.iota(jnp.int32, L)
                return tbl, idx, gidx, hit

            def proc_b(c, par):
                def group(g, _):
                    fl = fb[pl.ds(par * (CH // L) + g * L, L)]
                    base = par * CH + g * GRP
                    coff = c * CH + g * GRP

                    @pl.when(jnp.max(fl) == 0)
                    def _():
                        for v in range(L):
                            tbl, idx, gidx, hit = vstep_b(coff, base, v)
                            old = plsc.load_gather(tbl, [idx])
                            plsc.store_scatter(
                                tbl, [idx], jnp.minimum(old, gidx),
                                mask=hit)

                    @pl.when(jnp.max(fl) != 0)
                    def _():
                        for v in range(L):
                            tbl, idx, gidx, hit = vstep_b(coff, base, v)
                            _scatter_min_retry(tbl, idx, gidx, hit)

                    return 0

                lax.fori_loop(0, CH // GRP, group, 0)

            stream_loop(pw // CH, issue_b, wait_b, proc_b)

            def merge_min(i, _):
                sl = pl.ds(i * L, L)
                waminta[sl] = jnp.minimum(waminta[sl], wamintb[sl])
                return 0

            lax.fori_loop(0, SUP2 // L, merge_min, 0)
            pltpu.sync_copy(waminta, amin_hbm.at[img, cls])

        def sizes_task(img):
            _fill_f32(sumt, SMALL2, 0.0)
            irow = idss.at[img]
            one = jnp.full((L,), 1.0, jnp.float32)

            def issue(c, half):
                xcopy(irow, c, half, ib, si, CH).start()

            def wait(c, half):
                xcopy(irow, c, half, ib, si, CH).wait()

            def proc(c, par):
                def grp(g, _):
                    base = par * CH + g * GRP
                    for v in range(L):
                        idx = ib[pl.ds(base + v * L, L)]
                        plsc.addupdate_scatter(sumt, [idx], one)
                    return 0

                lax.fori_loop(0, CH // GRP, grp, 0)

            stream_loop(ps // CH, issue, wait, proc)
            pltpu.sync_copy(sumt, sizes_hbm.at[img])

        def kloop(k, _):
            task = wid + k * NTILES

            @pl.when(task < n_strong)
            def _():
                strong_task(task)

            @pl.when(jnp.logical_and(task >= n_strong,
                                     task < n_strong + n_weak))
            def _():
                weak_task(task - n_strong)

            @pl.when(jnp.logical_and(task >= n_strong + n_weak,
                                     task < n_tasks))
            def _():
                sizes_task(task - n_strong - n_weak)

            return 0

        lax.fori_loop(0, k_max, kloop, 0)

    return sc1


def _make_sc2(n, pw):
    n_tasks = n * C
    k_max = (n_tasks + NTILES - 1) // NTILES
    mesh = plsc.VectorSubcoreMesh(core_axis_name="c", subcore_axis_name="s")

    @functools.partial(
        pl.kernel,
        mesh=mesh,
        compiler_params=pltpu.CompilerParams(needs_layout_passes=False),
        out_type=(
            jax.ShapeDtypeStruct((n, C, L), jnp.float32),
            jax.ShapeDtypeStruct((n, C, L), jnp.float32),
        ),
        scratch_types=(
            pltpu.VMEM((pw,), jnp.int32),
            pltpu.VMEM((SMALL2,), jnp.float32),
            pltpu.VMEM((SMALL2,), jnp.float32),
            pltpu.VMEM((SMALL2,), jnp.float32),
            pltpu.VMEM((SUP2,), jnp.int32),
            pltpu.VMEM((SUP2,), jnp.int32),
            pltpu.VMEM((SUP,), jnp.int32),
            pltpu.VMEM((L,), jnp.float32),
            pltpu.VMEM((L,), jnp.float32),
        ),
    )
    def sc2(sum_hbm, max_hbm, sizes_hbm, amin_hbm, smallw_hbm, trg_hbm,
            lout, nout,
            swv, sumt, maxt, sizest, am0, amc, tgc, lbuf, nbuf):
        wid = lax.axis_index("s") * 2 + lax.axis_index("c")

        def task_body(t):
            img = t // C
            cls = t - img * C
            pltpu.sync_copy(smallw_hbm.at[img], swv)
            pltpu.sync_copy(sum_hbm.at[img, cls], sumt)
            pltpu.sync_copy(max_hbm.at[img, cls], maxt)
            pltpu.sync_copy(sizes_hbm.at[img], sizest)
            pltpu.sync_copy(amin_hbm.at[img, 0], am0)
            pltpu.sync_copy(amin_hbm.at[img, cls], amc)
            pltpu.sync_copy(trg_hbm.at[img, cls], tgc)

            def vbody(v, st):
                lacc, nacc = st
                sl = pl.ds(v * L, L)
                a0 = am0[sl]
                ac = amc[sl]
                tg = tgc[sl]
                pm = jnp.logical_and(a0 < pw, tg > 0)
                sel = plsc.load_gather(swv, [jnp.minimum(ac, pw - 1)])
                val = plsc.load_gather(sumt, [sel])
                w = jnp.exp(-plsc.load_gather(maxt, [sel]))
                sz = plsc.load_gather(sizest, [sel])
                lacc = lacc + jnp.where(pm, w * val, 0.0)
                nacc = nacc + jnp.where(pm, sz, 0.0)
                return lacc, nacc

            z = jnp.zeros((L,), jnp.float32)
            lacc, nacc = lax.fori_loop(0, SUP // L, vbody, (z, z))
            lbuf[...] = lacc
            nbuf[...] = nacc
            pltpu.sync_copy(lbuf, lout.at[img, cls])
            pltpu.sync_copy(nbuf, nout.at[img, cls])

        def kloop(k, _):
            task = wid + k * NTILES

            @pl.when(task < n_tasks)
            def _():
                task_body(task)

            return 0

        lax.fori_loop(0, k_max, kloop, 0)

    return sc2


def kernel(inputs, inputs_weak, targets, spmasks, spmasks_weak,
           superpixels, superpixels_weak, superpixel_smalls, spx_smalls_weak):
    n, c, h, w = inputs.shape
    _, _, hw, ww = inputs_weak.shape
    ps = h * w
    pw = hw * ww

    xs = inputs.reshape(n, c, ps)
    xw = inputs_weak.reshape(n, c, pw)
    mask_s = spmasks.reshape(n, ps).astype(jnp.int32)
    mask_w = spmasks_weak.reshape(n, pw).astype(jnp.int32)
    ids_s = superpixel_smalls.reshape(n, ps).astype(jnp.int32)
    sup_w = superpixels_weak.reshape(n, pw).astype(jnp.int32)
    small_w = spx_smalls_weak.reshape(n, pw).astype(jnp.int32)
    trg = targets[:, :, :C].transpose(0, 2, 1).astype(jnp.int32)

    enc_s, ids2_s = _prep(xs, mask_s, ids_s, SMALL, TCB)
    enc_w, ids2_w = _prep(xw, mask_w, sup_w, SUP, TCB)
    flags_s = _flags(ids2_s, CH)
    flags_w = _flags(ids2_w, CH)

    sum_t, max_t, sizes_t, amin_t = _make_sc1(n, ps, pw)(
        enc_s, ids2_s, flags_s, enc_w, ids2_w, flags_w)
    lparts, nparts = _make_sc2(n, pw)(
        sum_t, max_t, sizes_t, amin_t, small_w, trg)

    loss = jnp.sum(lparts)
    num_valid = jnp.float32(1.0) + jnp.sum(nparts)
    return loss / num_valid


# speedup vs baseline: 1.0955x; 1.0685x over previous
import functools

import jax
import jax.numpy as jnp
from jax import lax
from jax.experimental import pallas as pl
from jax.experimental.pallas import tpu as pltpu
from jax.experimental.pallas import tpu_sc as plsc

C = 19
SUP = 2048
SMALL = 8192
SUP2 = SUP + 16
SMALL2 = SMALL + 16
L = 16
NTILES = 32
CH = 4096
TCB = 8192
GRP = 16 * L


def _prep_body(dead, x_ref, m_ref, i_ref, o_ref, oi_ref):
    x = x_ref[0]
    mask = m_ref[0, 0]
    ids = i_ref[0, 0]
    mx = jnp.max(x, axis=0, keepdims=True)
    lse = jnp.log(jnp.sum(jnp.exp(x - mx), axis=0, keepdims=True))
    mlse = jnp.where(mask > 0, mx + lse, 0.0)
    o_ref[0] = mlse - jnp.where(mask > 0, x, 1.0)
    lane = lax.broadcasted_iota(jnp.int32, ids.shape, 1) & (L - 1)
    oi_ref[0, 0] = jnp.where(mask > 0, ids, dead + lane)


def _prep(x, mask, ids, dead, blk):
    n, c, p = x.shape
    nb = p // blk
    mask4 = mask.reshape(n, nb, 1, blk)
    ids4 = ids.reshape(n, nb, 1, blk)
    out, out_ids = pl.pallas_call(
        functools.partial(_prep_body, dead),
        grid=(n, nb),
        in_specs=[
            pl.BlockSpec((1, c, blk), lambda i, j: (i, 0, j)),
            pl.BlockSpec((1, 1, 1, blk), lambda i, j: (i, j, 0, 0)),
            pl.BlockSpec((1, 1, 1, blk), lambda i, j: (i, j, 0, 0)),
        ],
        out_specs=[
            pl.BlockSpec((1, c, blk), lambda i, j: (i, 0, j)),
            pl.BlockSpec((1, 1, 1, blk), lambda i, j: (i, j, 0, 0)),
        ],
        out_shape=[
            jax.ShapeDtypeStruct((n, c, p), jnp.float32),
            jax.ShapeDtypeStruct((n, nb, 1, blk), jnp.int32),
        ],
    )(x, mask4, ids4)
    return out, out_ids.reshape(n, p)


def _flags_body(i_ref, o_ref):
    a = i_ref[0]
    dup = jnp.zeros(a.shape[1:], jnp.bool_)[None]
    for k in range(1, 9):
        dup = jnp.logical_or(dup, jnp.any(a == jnp.roll(a, k, axis=0),
                                          axis=0, keepdims=True))
    o_ref[0, 0] = dup.astype(jnp.int32)


def _flags(ids2, blk):
    n, p = ids2.shape
    g = p // L
    nb = g // blk
    ids_t = ids2.reshape(n, g, L).transpose(0, 2, 1)
    out = pl.pallas_call(
        _flags_body,
        grid=(n, nb),
        in_specs=[pl.BlockSpec((1, L, blk), lambda i, j: (i, 0, j))],
        out_specs=pl.BlockSpec((1, 1, 1, blk), lambda i, j: (i, j, 0, 0)),
        out_shape=jax.ShapeDtypeStruct((n, nb, 1, blk), jnp.int32),
    )(ids_t)
    return out.reshape(n, g)


def _fill_f32(ref, n, val):
    v = jnp.full((L,), val, jnp.float32)

    def body(i, _):
        ref[pl.ds(i * L, L)] = v
        return 0

    lax.fori_loop(0, n // L, body, 0)


def _fill_i32(ref, n, val):
    v = jnp.full((L,), val, jnp.int32)

    def body(i, _):
        ref[pl.ds(i * L, L)] = v
        return 0

    lax.fori_loop(0, n // L, body, 0)


def _scatter_min_retry(ref, idx, v, m0):
    def cond(st):
        return jnp.max(st[0].astype(jnp.int32)) > 0

    def body(st):
        pend, vv = st
        old = plsc.load_gather(ref, [idx])
        new = jnp.minimum(old, vv)
        plsc.store_scatter(ref, [idx], new, mask=pend)
        chk = plsc.load_gather(ref, [idx])
        return jnp.logical_and(pend, chk > new), vv

    lax.while_loop(cond, body, (m0, v))


def _make_sc1(n, ps, pw):
    n_strong = n * C
    n_weak = n * C
    n_tasks = n_strong + n_weak + n
    k_max = (n_tasks + NTILES - 1) // NTILES
    mesh = plsc.VectorSubcoreMesh(core_axis_name="c", subcore_axis_name="s")

    @functools.partial(
        pl.kernel,
        mesh=mesh,
        compiler_params=pltpu.CompilerParams(needs_layout_passes=False),
        out_type=(
            jax.ShapeDtypeStruct((n, C, SMALL2), jnp.float32),
            jax.ShapeDtypeStruct((n, C, SMALL2), jnp.float32),
            jax.ShapeDtypeStruct((n, SMALL2), jnp.float32),
            jax.ShapeDtypeStruct((n, C, SUP2), jnp.int32),
        ),
    scratch_types=(
            pltpu.VMEM((2 * CH,), jnp.float32),
            pltpu.VMEM((2 * CH,), jnp.int32),
            pltpu.VMEM((2 * (CH // L),), jnp.int32),
            pltpu.VMEM((pw,), jnp.float32),
            pltpu.VMEM((SMALL2,), jnp.float32),
            pltpu.VMEM((SMALL2,), jnp.float32),
            pltpu.VMEM((SMALL2,), jnp.float32),
            pltpu.VMEM((SUP2,), jnp.float32),
            pltpu.VMEM((SUP2,), jnp.float32),
            pltpu.VMEM((SUP2,), jnp.int32),
            pltpu.VMEM((SUP2,), jnp.int32),
            pltpu.SemaphoreType.DMA,
            pltpu.SemaphoreType.DMA,
            pltpu.SemaphoreType.DMA,
        ),
    )
    def sc1(xs, idss, fls, xw, supw, flw,
            sum_hbm, max_hbm, sizes_hbm, amin_hbm,
            xb, ib, fb, pcache, sumt, maxta, maxtb,
            wmaxta, wmaxtb, waminta, wamintb,
            sx, si, sf):
        wid = lax.axis_index("s") * 2 + lax.axis_index("c")

        def xcopy(src, c, half, buf, sem, width):
            return pltpu.make_async_copy(
                src.at[pl.ds(c * width, width)],
                buf.at[pl.ds(pl.multiple_of(half * width, 8), width)],
                sem)

        def stream_loop(nch, issue, wait, proc, scat_issue=None,
                        scat_wait=None):
            issue(0, 0)

            def cbody(c, _):
                par = lax.rem(c, 2)

                if scat_wait is not None:
                    @pl.when(c > 0)
                    def _():
                        scat_wait()

                @pl.when(c + 1 < nch)
                def _():
                    issue(c + 1, 1 - par)

                wait(c, par)
                if scat_issue is not None:
                    scat_issue(par)
                proc(c, par)
                return 0

            lax.fori_loop(0, nch, cbody, 0)
            if scat_wait is not None:
                scat_wait()

        def merge_min_f(ta, tb, nrows):
            def body(i, _):
                sl = pl.ds(i * L, L)
                ta[sl] = jnp.minimum(ta[sl], tb[sl])
                return 0

            lax.fori_loop(0, nrows // L, body, 0)

        zl = jnp.zeros((L,), jnp.int32)
        lane_lo = lax.iota(jnp.int32, L) < (L // 2)

        def half_dispatch(fl, fast, slow):
            flo = jnp.max(jnp.where(lane_lo, fl, zl))
            fhi = jnp.max(jnp.where(lane_lo, zl, fl))

            @pl.when(flo == 0)
            def _():
                for v in range(L // 2):
                    fast(v)

            @pl.when(flo != 0)
            def _():
                for v in range(L // 2):
                    slow(v)

            @pl.when(fhi == 0)
            def _():
                for v in range(L // 2, L):
                    fast(v)

            @pl.when(fhi != 0)
            def _():
                for v in range(L // 2, L):
                    slow(v)

        def strong_task(t):
            img = t // C
            cls = t - img * C
            _fill_f32(sumt, SMALL2, 0.0)
            _fill_f32(maxta, SMALL2, 1e30)
            _fill_f32(maxtb, SMALL2, 1e30)
            xrow = xs.at[img, cls]
            irow = idss.at[img]
            frow = fls.at[img]

            def issue(c, half):
                xcopy(xrow, c, half, xb, sx, CH).start()
                xcopy(irow, c, half, ib, si, CH).start()
                xcopy(frow, c, half, fb, sf, CH // L).start()

            def wait(c, half):
                xcopy(xrow, c, half, xb, sx, CH).wait()
                xcopy(irow, c, half, ib, si, CH).wait()
                xcopy(frow, c, half, fb, sf, CH // L).wait()

            def vstep(base, v):
                tbl = maxta if v % 2 == 0 else maxtb
                sl = pl.ds(base + v * L, L)
                x = xb[sl]
                idx = ib[sl]
                plsc.addupdate_scatter(sumt, [idx], x)
                old = plsc.load_gather(tbl, [idx])
                plsc.store_scatter(tbl, [idx], jnp.minimum(old, x))
                return tbl, idx, x

            def proc(c, par):
                def group(g, _):
                    fl = fb[pl.ds(par * (CH // L) + g * L, L)]
                    base = par * CH + g * GRP

                    def slow(v):
                        tbl, idx, x = vstep(base, v)
                        _scatter_min_retry(tbl, idx, x,
                                           jnp.ones((L,), jnp.bool_))

                    half_dispatch(fl, lambda v: vstep(base, v), slow)
                    return 0

                lax.fori_loop(0, CH // GRP, group, 0)

            stream_loop(ps // CH, issue, wait, proc)
            merge_min_f(maxta, maxtb, SMALL2)
            pltpu.sync_copy(sumt, sum_hbm.at[img, cls])
            pltpu.sync_copy(maxta, max_hbm.at[img, cls])

        def weak_task(t):
            img = t // C
            cls = t - img * C
            _fill_f32(wmaxta, SUP2, 1e30)
            _fill_f32(wmaxtb, SUP2, 1e30)
            _fill_i32(waminta, SUP2, pw)
            _fill_i32(wamintb, SUP2, pw)
            xrow = xw.at[img, cls]
            irow = supw.at[img]
            frow = flw.at[img]

            def issue_a(c, half):
                xcopy(xrow, c, half, xb, sx, CH).start()
                xcopy(irow, c, half, ib, si, CH).start()
                xcopy(frow, c, half, fb, sf, CH // L).start()

            def wait_a(c, half):
                xcopy(xrow, c, half, xb, sx, CH).wait()
                xcopy(irow, c, half, ib, si, CH).wait()
                xcopy(frow, c, half, fb, sf, CH // L).wait()

            def vstep_a(coff, base, v):
                tbl = wmaxta if v % 2 == 0 else wmaxtb
                sl = pl.ds(base + v * L, L)
                x = xb[sl]
                idx = ib[sl]
                pcache[pl.ds(coff + v * L, L)] = x
                old = plsc.load_gather(tbl, [idx])
                plsc.store_scatter(tbl, [idx], jnp.minimum(old, x))
                return tbl, idx, x

            def proc_a(c, par):
                def group(g, _):
                    fl = fb[pl.ds(par * (CH // L) + g * L, L)]
                    base = par * CH + g * GRP
                    coff = c * CH + g * GRP

                    def slow(v):
                        tbl, idx, x = vstep_a(coff, base, v)
                        _scatter_min_retry(tbl, idx, x,
                                           jnp.ones((L,), jnp.bool_))

                    half_dispatch(fl, lambda v: vstep_a(coff, base, v), slow)
                    return 0

                lax.fori_loop(0, CH // GRP, group, 0)

            stream_loop(pw // CH, issue_a, wait_a, proc_a)
            merge_min_f(wmaxta, wmaxtb, SUP2)

            def issue_b(c, half):
                xcopy(irow, c, half, ib, si, CH).start()
                xcopy(frow, c, half, fb, sf, CH // L).start()

            def wait_b(c, half):
                xcopy(irow, c, half, ib, si, CH).wait()
                xcopy(frow, c, half, fb, sf, CH // L).wait()

            def vstep_b(coff, base, v):
                tbl = waminta if v % 2 == 0 else wamintb
                x = pcache[pl.ds(coff + v * L, L)]
                idx = ib[pl.ds(base + v * L, L)]
                smin = plsc.load_gather(wmaxta, [idx])
                hit = x == smin
                gidx = coff + v * L + lax.iota(jnp.int32, L)
                return tbl, idx, gidx, hit

            def proc_b(c, par):
                def group(g, _):
                    fl = fb[pl.ds(par * (CH // L) + g * L, L)]
                    base = par * CH + g * GRP
                    coff = c * CH + g * GRP

                    def fast(v):
                        tbl, idx, gidx, hit = vstep_b(coff, base, v)
                        old = plsc.load_gather(tbl, [idx])
                        plsc.store_scatter(
                            tbl, [idx], jnp.minimum(old, gidx), mask=hit)

                    def slow(v):
                        tbl, idx, gidx, hit = vstep_b(coff, base, v)
                        _scatter_min_retry(tbl, idx, gidx, hit)

                    half_dispatch(fl, fast, slow)
                    return 0

                lax.fori_loop(0, CH // GRP, group, 0)

            stream_loop(pw // CH, issue_b, wait_b, proc_b)

            def merge_min(i, _):
                sl = pl.ds(i * L, L)
                waminta[sl] = jnp.minimum(waminta[sl], wamintb[sl])
                return 0

            lax.fori_loop(0, SUP2 // L, merge_min, 0)
            pltpu.sync_copy(waminta, amin_hbm.at[img, cls])

        def sizes_task(img):
            _fill_f32(sumt, SMALL2, 0.0)
            irow = idss.at[img]
            one = jnp.full((L,), 1.0, jnp.float32)

            def issue(c, half):
                xcopy(irow, c, half, ib, si, CH).start()

            def wait(c, half):
                xcopy(irow, c, half, ib, si, CH).wait()

            def proc(c, par):
                def grp(g, _):
                    base = par * CH + g * GRP
                    for v in range(L):
                        idx = ib[pl.ds(base + v * L, L)]
                        plsc.addupdate_scatter(sumt, [idx], one)
                    return 0

                lax.fori_loop(0, CH // GRP, grp, 0)

            stream_loop(ps // CH, issue, wait, proc)
            pltpu.sync_copy(sumt, sizes_hbm.at[img])

        def kloop(k, _):
            task = wid + k * NTILES

            @pl.when(task < n_strong)
            def _():
                strong_task(task)

            @pl.when(jnp.logical_and(task >= n_strong,
                                     task < n_strong + n_weak))
            def _():
                weak_task(task - n_strong)

            @pl.when(jnp.logical_and(task >= n_strong + n_weak,
                                     task < n_tasks))
            def _():
                sizes_task(task - n_strong - n_weak)

            return 0

        lax.fori_loop(0, k_max, kloop, 0)

    return sc1


def _make_sc2(n, pw):
    n_tasks = n * C
    k_max = (n_tasks + NTILES - 1) // NTILES
    mesh = plsc.VectorSubcoreMesh(core_axis_name="c", subcore_axis_name="s")

    @functools.partial(
        pl.kernel,
        mesh=mesh,
        compiler_params=pltpu.CompilerParams(needs_layout_passes=False),
        out_type=(
            jax.ShapeDtypeStruct((n, C, L), jnp.float32),
            jax.ShapeDtypeStruct((n, C, L), jnp.float32),
        ),
        scratch_types=(
            pltpu.VMEM((pw,), jnp.int32),
            pltpu.VMEM((SMALL2,), jnp.float32),
            pltpu.VMEM((SMALL2,), jnp.float32),
            pltpu.VMEM((SMALL2,), jnp.float32),
            pltpu.VMEM((SUP2,), jnp.int32),
            pltpu.VMEM((SUP2,), jnp.int32),
            pltpu.VMEM((SUP,), jnp.int32),
            pltpu.VMEM((L,), jnp.float32),
            pltpu.VMEM((L,), jnp.float32),
        ),
    )
    def sc2(sum_hbm, max_hbm, sizes_hbm, amin_hbm, smallw_hbm, trg_hbm,
            lout, nout,
            swv, sumt, maxt, sizest, am0, amc, tgc, lbuf, nbuf):
        wid = lax.axis_index("s") * 2 + lax.axis_index("c")

        def task_body(t):
            img = t // C
            cls = t - img * C
            pltpu.sync_copy(smallw_hbm.at[img], swv)
            pltpu.sync_copy(sum_hbm.at[img, cls], sumt)
            pltpu.sync_copy(max_hbm.at[img, cls], maxt)
            pltpu.sync_copy(sizes_hbm.at[img], sizest)
            pltpu.sync_copy(amin_hbm.at[img, 0], am0)
            pltpu.sync_copy(amin_hbm.at[img, cls], amc)
            pltpu.sync_copy(trg_hbm.at[img, cls], tgc)

            def vbody(v, st):
                lacc, nacc = st
                sl = pl.ds(v * L, L)
                a0 = am0[sl]
                ac = amc[sl]
                tg = tgc[sl]
                pm = jnp.logical_and(a0 < pw, tg > 0)
                sel = plsc.load_gather(swv, [jnp.minimum(ac, pw - 1)])
                val = plsc.load_gather(sumt, [sel])
                w = jnp.exp(-plsc.load_gather(maxt, [sel]))
                sz = plsc.load_gather(sizest, [sel])
                lacc = lacc + jnp.where(pm, w * val, 0.0)
                nacc = nacc + jnp.where(pm, sz, 0.0)
                return lacc, nacc

            z = jnp.zeros((L,), jnp.float32)
            lacc, nacc = lax.fori_loop(0, SUP // L, vbody, (z, z))
            lbuf[...] = lacc
            nbuf[...] = nacc
            pltpu.sync_copy(lbuf, lout.at[img, cls])
            pltpu.sync_copy(nbuf, nout.at[img, cls])

        def kloop(k, _):
            task = wid + k * NTILES

            @pl.when(task < n_tasks)
            def _():
                task_body(task)

            return 0

        lax.fori_loop(0, k_max, kloop, 0)

    return sc2


def kernel(inputs, inputs_weak, targets, spmasks, spmasks_weak,
           superpixels, superpixels_weak, superpixel_smalls, spx_smalls_weak):
    n, c, h, w = inputs.shape
    _, _, hw, ww = inputs_weak.shape
    ps = h * w
    pw = hw * ww

    xs = inputs.reshape(n, c, ps)
    xw = inputs_weak.reshape(n, c, pw)
    mask_s = spmasks.reshape(n, ps).astype(jnp.int32)
    mask_w = spmasks_weak.reshape(n, pw).astype(jnp.int32)
    ids_s = superpixel_smalls.reshape(n, ps).astype(jnp.int32)
    sup_w = superpixels_weak.reshape(n, pw).astype(jnp.int32)
    small_w = spx_smalls_weak.reshape(n, pw).astype(jnp.int32)
    trg = targets[:, :, :C].transpose(0, 2, 1).astype(jnp.int32)

    enc_s, ids2_s = _prep(xs, mask_s, ids_s, SMALL, TCB)
    enc_w, ids2_w = _prep(xw, mask_w, sup_w, SUP, TCB)
    flags_s = _flags(ids2_s, CH)
    flags_w = _flags(ids2_w, CH)

    sum_t, max_t, sizes_t, amin_t = _make_sc1(n, ps, pw)(
        enc_s, ids2_s, flags_s, enc_w, ids2_w, flags_w)
    lparts, nparts = _make_sc2(n, pw)(
        sum_t, max_t, sizes_t, amin_t, small_w, trg)

    loss = jnp.sum(lparts)
    num_valid = jnp.float32(1.0) + jnp.sum(nparts)
    return loss / num_valid
